# baseline (speedup 1.0000x reference)
import functools

import jax
import jax.numpy as jnp
from jax import lax
from jax.experimental import pallas as pl
from jax.experimental.pallas import tpu as pltpu
from jax.experimental.pallas import tpu_sc as plsc

VOCAB = 100000
EMBED_DIM = 16
BATCH = 1024
HIST = 200

_C0 = 128
_C1 = HIST - _C0


def _sc_gather_mean(x, emb, row0, nrows):
    info = plsc.get_sparse_core_info()
    nc, ns = info.num_cores, info.num_subcores
    nw = nc * ns
    bpw = nrows // nw

    mesh = plsc.VectorSubcoreMesh(core_axis_name="c", subcore_axis_name="s")

    @functools.partial(
        pl.kernel,
        mesh=mesh,
        out_type=jax.ShapeDtypeStruct((nrows, EMBED_DIM), jnp.float32),
        scratch_types=[
            pltpu.VMEM((bpw, HIST), jnp.int32),
            pltpu.VMEM((4, HIST, EMBED_DIM), jnp.float32),
            pltpu.VMEM((bpw, EMBED_DIM), jnp.float32),
            pltpu.SemaphoreType.DMA((4,)),
        ],
        compiler_params=pltpu.CompilerParams(use_tc_tiling_on_sc=False),
    )
    def sc_kernel(x_hbm, emb_hbm, h_hbm, xv, rows, hv, sems):
        wid = lax.axis_index("s") * nc + lax.axis_index("c")
        base = wid * bpw
        pltpu.sync_copy(x_hbm.at[pl.ds(row0 + base, bpw)], xv)

        def gather_row(r, s):
            pltpu.make_async_copy(
                emb_hbm.at[xv.at[r, pl.ds(0, _C0)]],
                rows.at[s, pl.ds(0, _C0)], sems.at[s]).start()
            pltpu.make_async_copy(
                emb_hbm.at[xv.at[r, pl.ds(_C0, _C1)]],
                rows.at[s, pl.ds(_C0, _C1)], sems.at[s]).start()

        def wait_row(r, s):
            pltpu.make_async_copy(
                emb_hbm.at[xv.at[r, pl.ds(0, _C0)]],
                rows.at[s, pl.ds(0, _C0)], sems.at[s]).wait()
            pltpu.make_async_copy(
                emb_hbm.at[xv.at[r, pl.ds(_C0, _C1)]],
                rows.at[s, pl.ds(_C0, _C1)], sems.at[s]).wait()

        gather_row(0, 0)
        gather_row(1, 1)
        gather_row(2, 2)

        def row_body(i, carry):
            slot = lax.rem(i, 4)

            @pl.when(i + 3 < bpw)
            def _prefetch():
                gather_row(i + 3, lax.rem(i + 3, 4))

            wait_row(i, slot)

            def red(j, acc):
                a = acc
                for u in range(8):
                    a = a + rows[slot, 8 * j + u]
                return a

            acc = lax.fori_loop(0, HIST // 8, red, jnp.zeros((EMBED_DIM,), jnp.float32))
            hv[i] = acc * jnp.float32(1.0 / HIST)
            return carry

        lax.fori_loop(0, bpw, row_body, 0)
        pltpu.sync_copy(hv, h_hbm.at[pl.ds(base, bpw)])

    return sc_kernel(x, emb)


def _tc_matmul(h, W, b2d, row0, nrows, prev=None):
    mb = 8
    nsteps = nrows // mb
    nbuf = 4

    def body(*refs):
        if prev is None:
            h_ref, w_ref, b_ref, o_ref, buf, sems = refs
        else:
            h_ref, w_ref, b_ref, _prev_ref, o_ref, buf, sems = refs
        j = pl.program_id(0)
        slot = lax.rem(j, nbuf)

        @pl.when(j >= nbuf)
        def _wait_old():
            pltpu.make_async_copy(
                buf.at[slot],
                o_ref.at[pl.ds(row0 + (j - nbuf) * mb, mb)],
                sems.at[slot],
            ).wait()

        buf[slot] = (
            jnp.dot(h_ref[...], w_ref[...], preferred_element_type=jnp.float32)
            + b_ref[...]
        )
        pltpu.make_async_copy(
            buf.at[slot], o_ref.at[pl.ds(row0 + j * mb, mb)], sems.at[slot]
        ).start()

        @pl.when(j == nsteps - 1)
        def _drain():
            for k in range(1, nbuf + 1):
                s = lax.rem(j - nbuf + k + nbuf, nbuf)
                pltpu.make_async_copy(
                    buf.at[s],
                    o_ref.at[pl.ds(row0 + (j - nbuf + k) * mb, mb)],
                    sems.at[s],
                ).wait()

    in_specs = [
        pl.BlockSpec((mb, EMBED_DIM), lambda j: (j, 0)),
        pl.BlockSpec((EMBED_DIM, VOCAB), lambda j: (0, 0)),
        pl.BlockSpec((1, VOCAB), lambda j: (0, 0)),
    ]
    args = [h, W, b2d]
    aliases = {}
    if prev is not None:
        in_specs.append(pl.BlockSpec(memory_space=pl.ANY))
        args.append(prev)
        aliases = {3: 0}

    return pl.pallas_call(
        body,
        grid=(nsteps,),
        in_specs=in_specs,
        out_specs=pl.BlockSpec(memory_space=pl.ANY),
        out_shape=jax.ShapeDtypeStruct((BATCH, VOCAB), jnp.float32),
        input_output_aliases=aliases,
        scratch_shapes=[
            pltpu.VMEM((nbuf, mb, VOCAB), jnp.float32),
            pltpu.SemaphoreType.DMA((nbuf,)),
        ],
    )(*args)


def kernel(x, emb, W, b):
    half = BATCH // 2
    h0 = _sc_gather_mean(x, emb, 0, half)
    h1 = _sc_gather_mean(x, emb, half, half)
    b2d = b.reshape(1, VOCAB)
    out = _tc_matmul(h0, W, b2d, 0, half)
    out = _tc_matmul(h1, W, b2d, half, half, prev=out)
    return out


# speedup vs baseline: 1.0189x; 1.0189x over previous
import functools

import jax
import jax.numpy as jnp
from jax import lax
from jax.experimental import pallas as pl
from jax.experimental.pallas import tpu as pltpu
from jax.experimental.pallas import tpu_sc as plsc

VOCAB = 100000
EMBED_DIM = 16
BATCH = 1024
HIST = 200

_C0 = 128
_C1 = HIST - _C0


def _sc_gather_mean(x, emb, row0, nrows):
    info = plsc.get_sparse_core_info()
    nc, ns = info.num_cores, info.num_subcores
    nw = nc * ns
    bpw = nrows // nw

    mesh = plsc.VectorSubcoreMesh(core_axis_name="c", subcore_axis_name="s")

    @functools.partial(
        pl.kernel,
        mesh=mesh,
        out_type=jax.ShapeDtypeStruct((nrows, EMBED_DIM), jnp.float32),
        scratch_types=[
            pltpu.VMEM((bpw, HIST), jnp.int32),
            pltpu.VMEM((6, HIST, EMBED_DIM), jnp.float32),
            pltpu.VMEM((bpw, EMBED_DIM), jnp.float32),
            pltpu.SemaphoreType.DMA((6,)),
        ],
        compiler_params=pltpu.CompilerParams(use_tc_tiling_on_sc=False),
    )
    def sc_kernel(x_hbm, emb_hbm, h_hbm, xv, rows, hv, sems):
        wid = lax.axis_index("s") * nc + lax.axis_index("c")
        base = wid * bpw
        pltpu.sync_copy(x_hbm.at[pl.ds(row0 + base, bpw)], xv)

        def gather_row(r, s):
            pltpu.make_async_copy(
                emb_hbm.at[xv.at[r, pl.ds(0, _C0)]],
                rows.at[s, pl.ds(0, _C0)], sems.at[s]).start()
            pltpu.make_async_copy(
                emb_hbm.at[xv.at[r, pl.ds(_C0, _C1)]],
                rows.at[s, pl.ds(_C0, _C1)], sems.at[s]).start()

        def wait_row(r, s):
            pltpu.make_async_copy(
                emb_hbm.at[xv.at[r, pl.ds(0, _C0)]],
                rows.at[s, pl.ds(0, _C0)], sems.at[s]).wait()
            pltpu.make_async_copy(
                emb_hbm.at[xv.at[r, pl.ds(_C0, _C1)]],
                rows.at[s, pl.ds(_C0, _C1)], sems.at[s]).wait()

        for r in range(5):
            gather_row(r, r)

        def row_body(i, carry):
            slot = lax.rem(i, 6)

            @pl.when(i + 5 < bpw)
            def _prefetch():
                gather_row(i + 5, lax.rem(i + 5, 6))

            wait_row(i, slot)

            def red(j, acc):
                a = acc
                for u in range(8):
                    a = a + rows[slot, 8 * j + u]
                return a

            acc = lax.fori_loop(0, HIST // 8, red, jnp.zeros((EMBED_DIM,), jnp.float32))
            hv[i] = acc * jnp.float32(1.0 / HIST)
            return carry

        lax.fori_loop(0, bpw, row_body, 0)
        pltpu.sync_copy(hv, h_hbm.at[pl.ds(base, bpw)])

    return sc_kernel(x, emb)


def _tc_matmul(h, W, b2d, row0, nrows, prev=None):
    mb = 8
    nsteps = nrows // mb
    nbuf = 4

    def body(*refs):
        if prev is None:
            h_ref, w_ref, b_ref, o_ref, buf, sems = refs
        else:
            h_ref, w_ref, b_ref, _prev_ref, o_ref, buf, sems = refs
        j = pl.program_id(0)
        slot = lax.rem(j, nbuf)

        @pl.when(j >= nbuf)
        def _wait_old():
            pltpu.make_async_copy(
                buf.at[slot],
                o_ref.at[pl.ds(row0 + (j - nbuf) * mb, mb)],
                sems.at[slot],
            ).wait()

        buf[slot] = (
            jnp.dot(h_ref[...], w_ref[...], preferred_element_type=jnp.float32)
            + b_ref[...]
        )
        pltpu.make_async_copy(
            buf.at[slot], o_ref.at[pl.ds(row0 + j * mb, mb)], sems.at[slot]
        ).start()

        @pl.when(j == nsteps - 1)
        def _drain():
            for k in range(1, nbuf + 1):
                s = lax.rem(j - nbuf + k + nbuf, nbuf)
                pltpu.make_async_copy(
                    buf.at[s],
                    o_ref.at[pl.ds(row0 + (j - nbuf + k) * mb, mb)],
                    sems.at[s],
                ).wait()

    in_specs = [
        pl.BlockSpec((mb, EMBED_DIM), lambda j: (j, 0)),
        pl.BlockSpec((EMBED_DIM, VOCAB), lambda j: (0, 0)),
        pl.BlockSpec((1, VOCAB), lambda j: (0, 0)),
    ]
    args = [h, W, b2d]
    aliases = {}
    if prev is not None:
        in_specs.append(pl.BlockSpec(memory_space=pl.ANY))
        args.append(prev)
        aliases = {3: 0}

    return pl.pallas_call(
        body,
        grid=(nsteps,),
        in_specs=in_specs,
        out_specs=pl.BlockSpec(memory_space=pl.ANY),
        out_shape=jax.ShapeDtypeStruct((BATCH, VOCAB), jnp.float32),
        input_output_aliases=aliases,
        scratch_shapes=[
            pltpu.VMEM((nbuf, mb, VOCAB), jnp.float32),
            pltpu.SemaphoreType.DMA((nbuf,)),
        ],
    )(*args)


def kernel(x, emb, W, b):
    h = _sc_gather_mean(x, emb, 0, BATCH)
    return _tc_matmul(h, W, b.reshape(1, VOCAB), 0, BATCH)


# speedup vs baseline: 1.0213x; 1.0023x over previous
import functools

import jax
import jax.numpy as jnp
from jax import lax
from jax.experimental import pallas as pl
from jax.experimental.pallas import tpu as pltpu
from jax.experimental.pallas import tpu_sc as plsc

VOCAB = 100000
EMBED_DIM = 16
BATCH = 1024
HIST = 200

_C0 = 128
_C1 = HIST - _C0


def _sc_gather_mean(x, emb, row0, nrows):
    info = plsc.get_sparse_core_info()
    nc, ns = info.num_cores, info.num_subcores
    nw = nc * ns
    bpw = nrows // nw

    mesh = plsc.VectorSubcoreMesh(core_axis_name="c", subcore_axis_name="s")

    @functools.partial(
        pl.kernel,
        mesh=mesh,
        out_type=jax.ShapeDtypeStruct((nrows, EMBED_DIM), jnp.float32),
        scratch_types=[
            pltpu.VMEM((bpw * HIST,), jnp.int32),
            pltpu.VMEM((6, HIST, EMBED_DIM), jnp.float32),
            pltpu.VMEM((bpw, EMBED_DIM), jnp.float32),
            pltpu.SemaphoreType.DMA((6,)),
        ],
        compiler_params=pltpu.CompilerParams(use_tc_tiling_on_sc=False),
    )
    def sc_kernel(x_hbm, emb_hbm, h_hbm, xv, rows, hv, sems):
        wid = lax.axis_index("s") * nc + lax.axis_index("c")
        base = wid * bpw
        pltpu.sync_copy(x_hbm.at[pl.ds((row0 + base) * HIST, bpw * HIST)], xv)

        def gather_row(r, s):
            pltpu.make_async_copy(
                emb_hbm.at[xv.at[pl.ds(r * HIST, _C0)]],
                rows.at[s, pl.ds(0, _C0)], sems.at[s]).start()
            pltpu.make_async_copy(
                emb_hbm.at[xv.at[pl.ds(r * HIST + _C0, _C1)]],
                rows.at[s, pl.ds(_C0, _C1)], sems.at[s]).start()

        def wait_row(r, s):
            pltpu.make_async_copy(
                emb_hbm.at[xv.at[pl.ds(r * HIST, _C0)]],
                rows.at[s, pl.ds(0, _C0)], sems.at[s]).wait()
            pltpu.make_async_copy(
                emb_hbm.at[xv.at[pl.ds(r * HIST + _C0, _C1)]],
                rows.at[s, pl.ds(_C0, _C1)], sems.at[s]).wait()

        for r in range(5):
            gather_row(r, r)

        def row_body(i, carry):
            slot = lax.rem(i, 6)

            @pl.when(i + 5 < bpw)
            def _prefetch():
                gather_row(i + 5, lax.rem(i + 5, 6))

            wait_row(i, slot)

            def red(j, acc):
                a = acc
                for u in range(8):
                    a = a + rows[slot, 8 * j + u]
                return a

            acc = lax.fori_loop(0, HIST // 8, red, jnp.zeros((EMBED_DIM,), jnp.float32))
            hv[i] = acc * jnp.float32(1.0 / HIST)
            return carry

        lax.fori_loop(0, bpw, row_body, 0)
        pltpu.sync_copy(hv, h_hbm.at[pl.ds(base, bpw)])

    return sc_kernel(x, emb)


def _tc_matmul(h, W, b2d, row0, nrows, prev=None):
    mb = 8
    nsteps = nrows // mb
    nbuf = 4

    def body(*refs):
        if prev is None:
            h_ref, w_ref, b_ref, o_ref, buf, sems = refs
        else:
            h_ref, w_ref, b_ref, _prev_ref, o_ref, buf, sems = refs
        j = pl.program_id(0)
        slot = lax.rem(j, nbuf)

        @pl.when(j >= nbuf)
        def _wait_old():
            pltpu.make_async_copy(
                buf.at[slot],
                o_ref.at[pl.ds(row0 + (j - nbuf) * mb, mb)],
                sems.at[slot],
            ).wait()

        buf[slot] = (
            jnp.dot(h_ref[...], w_ref[...], preferred_element_type=jnp.float32)
            + b_ref[...]
        )
        pltpu.make_async_copy(
            buf.at[slot], o_ref.at[pl.ds(row0 + j * mb, mb)], sems.at[slot]
        ).start()

        @pl.when(j == nsteps - 1)
        def _drain():
            for k in range(1, nbuf + 1):
                s = lax.rem(j - nbuf + k + nbuf, nbuf)
                pltpu.make_async_copy(
                    buf.at[s],
                    o_ref.at[pl.ds(row0 + (j - nbuf + k) * mb, mb)],
                    sems.at[s],
                ).wait()

    in_specs = [
        pl.BlockSpec((mb, EMBED_DIM), lambda j: (j, 0)),
        pl.BlockSpec((EMBED_DIM, VOCAB), lambda j: (0, 0)),
        pl.BlockSpec((1, VOCAB), lambda j: (0, 0)),
    ]
    args = [h, W, b2d]
    aliases = {}
    if prev is not None:
        in_specs.append(pl.BlockSpec(memory_space=pl.ANY))
        args.append(prev)
        aliases = {3: 0}

    return pl.pallas_call(
        body,
        grid=(nsteps,),
        in_specs=in_specs,
        out_specs=pl.BlockSpec(memory_space=pl.ANY),
        out_shape=jax.ShapeDtypeStruct((BATCH, VOCAB), jnp.float32),
        input_output_aliases=aliases,
        scratch_shapes=[
            pltpu.VMEM((nbuf, mb, VOCAB), jnp.float32),
            pltpu.SemaphoreType.DMA((nbuf,)),
        ],
    )(*args)


def kernel(x, emb, W, b):
    h = _sc_gather_mean(x.reshape(BATCH * HIST), emb, 0, BATCH)
    return _tc_matmul(h, W, b.reshape(1, VOCAB), 0, BATCH)
